# baseline (speedup 1.0000x reference)
import functools

import jax
import jax.numpy as jnp
from jax.experimental import pallas as pl
from jax.experimental.pallas import tpu as pltpu

_B, _T, _D, _K, _S = 4, 2048, 1024, 64, 8
_R = _B * _S
_TH = _T // 8


def _accum(x, mu, mu_norm, b, s_acc):
    dot = jax.lax.dot_general(
        x, mu, (((1,), (1,)), ((), ())),
        preferred_element_type=jnp.float32)
    x_norm = jnp.sqrt(jnp.sum(x * x, axis=1, keepdims=True))
    cos = dot / jnp.maximum(x_norm * mu_norm[None, :], 1e-8)
    m = jnp.max(cos, axis=1, keepdims=True)
    e = jnp.exp(cos - m)
    p = e / jnp.sum(e, axis=1, keepdims=True)
    partial = jnp.sum(p, axis=0, keepdims=True)
    brow = jax.lax.broadcasted_iota(jnp.int32, (_R, 1), 0) // _S
    s_acc[...] += jnp.where(brow == b, partial, 0.0)


def _router_body(xa_ref, xb_ref, xc_ref, xd_ref, xe_ref, xf_ref, xg_ref, xh_ref, mu_ref, ls_ref, eps_ref, out_ref, s_acc):
    b = pl.program_id(0)

    @pl.when(b == 0)
    def _init():
        s_acc[...] = jnp.zeros_like(s_acc)

    mu = mu_ref[...]
    mu_norm = jnp.sqrt(jnp.sum(mu * mu, axis=1))
    _accum(xa_ref[0], mu, mu_norm, b, s_acc)
    _accum(xb_ref[0], mu, mu_norm, b, s_acc)
    _accum(xc_ref[0], mu, mu_norm, b, s_acc)
    _accum(xd_ref[0], mu, mu_norm, b, s_acc)
    _accum(xe_ref[0], mu, mu_norm, b, s_acc)
    _accum(xf_ref[0], mu, mu_norm, b, s_acc)
    _accum(xg_ref[0], mu, mu_norm, b, s_acc)
    _accum(xh_ref[0], mu, mu_norm, b, s_acc)

    @pl.when(b == _B - 1)
    def _finish():
        s = s_acc[...]
        sm = jnp.max(s, axis=1, keepdims=True)
        se = jnp.exp(s - sm)
        r = se / jnp.sum(se, axis=1, keepdims=True)
        iota_k = jax.lax.broadcasted_iota(jnp.int32, (_R, _K), 1)
        jrow = jax.lax.broadcasted_iota(jnp.int32, (_R, 1), 0) % _S
        oh_sel = jnp.zeros((_R, _K), jnp.float32)
        rr = r
        for j in range(_S):
            mj = jnp.max(rr, axis=1, keepdims=True)
            idxj = jnp.min(jnp.where(rr == mj, iota_k, _K), axis=1,
                           keepdims=True)
            oh = iota_k == idxj
            oh_sel = jnp.where(jrow == j, oh.astype(jnp.float32), oh_sel)
            rr = jnp.where(oh, -1.0, rr)
        mu_sel = jax.lax.dot_general(
            oh_sel, mu_ref[...], (((1,), (0,)), ((), ())),
            preferred_element_type=jnp.float32,
            precision=jax.lax.Precision.HIGHEST)
        ls_sel = jax.lax.dot_general(
            oh_sel, ls_ref[...], (((1,), (0,)), ((), ())),
            preferred_element_type=jnp.float32,
            precision=jax.lax.Precision.HIGHEST)
        out_ref[...] = mu_sel + jnp.exp(ls_sel) * eps_ref[...]


@jax.jit
def _run(x, mu, log_sigma, eps):
    out = pl.pallas_call(
        _router_body,
        grid=(_B,),
        in_specs=[
            pl.BlockSpec((1, _TH, _D), lambda b: (b, 0, 0)),
            pl.BlockSpec((1, _TH, _D), lambda b: (b, 1, 0)),
            pl.BlockSpec((1, _TH, _D), lambda b: (b, 2, 0)),
            pl.BlockSpec((1, _TH, _D), lambda b: (b, 3, 0)),
            pl.BlockSpec((1, _TH, _D), lambda b: (b, 4, 0)),
            pl.BlockSpec((1, _TH, _D), lambda b: (b, 5, 0)),
            pl.BlockSpec((1, _TH, _D), lambda b: (b, 6, 0)),
            pl.BlockSpec((1, _TH, _D), lambda b: (b, 7, 0)),
            pl.BlockSpec((_K, _D), lambda b: (0, 0)),
            pl.BlockSpec((_K, _D), lambda b: (0, 0)),
            pl.BlockSpec((_R, _D), lambda b: (0, 0)),
        ],
        out_specs=pl.BlockSpec((_R, _D), lambda b: (0, 0)),
        out_shape=jax.ShapeDtypeStruct((_R, _D), jnp.float32),
        scratch_shapes=[pltpu.VMEM((_R, _K), jnp.float32)],
        compiler_params=pltpu.CompilerParams(
            dimension_semantics=("arbitrary",)),
    )(x, x, x, x, x, x, x, x, mu, log_sigma, eps)
    return out.reshape(_B, _S, _D)


def kernel(x, mu, log_sigma, n_slots):
    eps = jax.random.normal(jax.random.key(42), (_R, _D), jnp.float32)
    return _run(x, mu, log_sigma, eps)


# speedup vs baseline: 1.0242x; 1.0242x over previous
import functools

import jax
import jax.numpy as jnp
from jax.experimental import pallas as pl
from jax.experimental.pallas import tpu as pltpu

_B, _T, _D, _K, _S = 4, 2048, 1024, 64, 8
_R = _B * _S
_TH = _T // 4


def _accum(x, mu, mu_norm, b, s_acc):
    dot = jax.lax.dot_general(
        x, mu, (((1,), (1,)), ((), ())),
        preferred_element_type=jnp.float32)
    x_norm = jnp.sqrt(jnp.sum(x * x, axis=1, keepdims=True))
    cos = dot / jnp.maximum(x_norm * mu_norm[None, :], 1e-8)
    m = jnp.max(cos, axis=1, keepdims=True)
    e = jnp.exp(cos - m)
    p = e / jnp.sum(e, axis=1, keepdims=True)
    partial = jnp.sum(p, axis=0, keepdims=True)
    brow = jax.lax.broadcasted_iota(jnp.int32, (_R, 1), 0) // _S
    s_acc[...] += jnp.where(brow == b, partial, 0.0)


def _router_body(xa_ref, xb_ref, xc_ref, xd_ref, mu_ref, ls_ref, eps_ref, out_ref, s_acc):
    b = pl.program_id(0)

    @pl.when(b == 0)
    def _init():
        s_acc[...] = jnp.zeros_like(s_acc)

    mu = mu_ref[...]
    mu_norm = jnp.sqrt(jnp.sum(mu * mu, axis=1))
    _accum(xa_ref[0], mu, mu_norm, b, s_acc)
    _accum(xb_ref[0], mu, mu_norm, b, s_acc)
    _accum(xc_ref[0], mu, mu_norm, b, s_acc)
    _accum(xd_ref[0], mu, mu_norm, b, s_acc)

    @pl.when(b == _B - 1)
    def _finish():
        s = s_acc[...]
        sm = jnp.max(s, axis=1, keepdims=True)
        se = jnp.exp(s - sm)
        r = se / jnp.sum(se, axis=1, keepdims=True)
        iota_k = jax.lax.broadcasted_iota(jnp.int32, (_R, _K), 1)
        jrow = jax.lax.broadcasted_iota(jnp.int32, (_R, 1), 0) % _S
        oh_sel = jnp.zeros((_R, _K), jnp.float32)
        rr = r
        for j in range(_S):
            mj = jnp.max(rr, axis=1, keepdims=True)
            idxj = jnp.min(jnp.where(rr == mj, iota_k, _K), axis=1,
                           keepdims=True)
            oh = iota_k == idxj
            oh_sel = jnp.where(jrow == j, oh.astype(jnp.float32), oh_sel)
            rr = jnp.where(oh, -1.0, rr)
        mu_sel = jax.lax.dot_general(
            oh_sel, mu_ref[...], (((1,), (0,)), ((), ())),
            preferred_element_type=jnp.float32,
            precision=jax.lax.Precision.HIGHEST)
        ls_sel = jax.lax.dot_general(
            oh_sel, ls_ref[...], (((1,), (0,)), ((), ())),
            preferred_element_type=jnp.float32,
            precision=jax.lax.Precision.HIGHEST)
        out_ref[...] = mu_sel + jnp.exp(ls_sel) * eps_ref[...]


@jax.jit
def _run(x, mu, log_sigma, eps):
    out = pl.pallas_call(
        _router_body,
        grid=(_B,),
        in_specs=[
            pl.BlockSpec((1, _TH, _D), lambda b: (b, 0, 0)),
            pl.BlockSpec((1, _TH, _D), lambda b: (b, 1, 0)),
            pl.BlockSpec((1, _TH, _D), lambda b: (b, 2, 0)),
            pl.BlockSpec((1, _TH, _D), lambda b: (b, 3, 0)),
            pl.BlockSpec((_K, _D), lambda b: (0, 0)),
            pl.BlockSpec((_K, _D), lambda b: (0, 0)),
            pl.BlockSpec((_R, _D), lambda b: (0, 0)),
        ],
        out_specs=pl.BlockSpec((_R, _D), lambda b: (0, 0)),
        out_shape=jax.ShapeDtypeStruct((_R, _D), jnp.float32),
        scratch_shapes=[pltpu.VMEM((_R, _K), jnp.float32)],
        compiler_params=pltpu.CompilerParams(
            dimension_semantics=("arbitrary",)),
    )(x, x, x, x, mu, log_sigma, eps)
    return out.reshape(_B, _S, _D)


def kernel(x, mu, log_sigma, n_slots):
    eps = jax.random.normal(jax.random.key(42), (_R, _D), jnp.float32)
    return _run(x, mu, log_sigma, eps)


# speedup vs baseline: 1.0276x; 1.0033x over previous
import jax
import jax.numpy as jnp
from jax.experimental import pallas as pl
from jax.experimental.pallas import tpu as pltpu

_B, _T, _D, _K, _S = 4, 2048, 1024, 64, 8
_R = _B * _S
_TH = _T // 4


def _accum(x, mu, mu_norm, b, s_acc):
    dot = jax.lax.dot_general(
        x, mu, (((1,), (1,)), ((), ())),
        preferred_element_type=jnp.float32)
    x_norm = jnp.sqrt(jnp.sum(x * x, axis=1, keepdims=True))
    cos = dot / jnp.maximum(x_norm * mu_norm[None, :], 1e-8)
    m = jnp.max(cos, axis=1, keepdims=True)
    e = jnp.exp(cos - m)
    p = e / jnp.sum(e, axis=1, keepdims=True)
    partial = jnp.sum(p, axis=0, keepdims=True)
    brow = jax.lax.broadcasted_iota(jnp.int32, (_R, 1), 0) // _S
    s_acc[...] += jnp.where(brow == b, partial, 0.0)


def _router_body(xa_ref, xb_ref, xc_ref, xd_ref, mu_ref, ls_ref, eps_ref, out_ref, s_acc):
    b = pl.program_id(0)

    @pl.when(b == 0)
    def _init():
        s_acc[...] = jnp.zeros_like(s_acc)

    mu = mu_ref[...]
    mu_norm = jnp.sqrt(jnp.sum(mu * mu, axis=1))
    _accum(xa_ref[0], mu, mu_norm, b, s_acc)
    _accum(xb_ref[0], mu, mu_norm, b, s_acc)
    _accum(xc_ref[0], mu, mu_norm, b, s_acc)
    _accum(xd_ref[0], mu, mu_norm, b, s_acc)

    @pl.when(b == _B - 1)
    def _finish():
        s = s_acc[...]
        sm = jnp.max(s, axis=1, keepdims=True)
        se = jnp.exp(s - sm)
        r = se / jnp.sum(se, axis=1, keepdims=True)
        iota_k = jax.lax.broadcasted_iota(jnp.int32, (_R, _K), 1)
        jrow = jax.lax.broadcasted_iota(jnp.int32, (_R, 1), 0) % _S
        oh_sel = jnp.zeros((_R, _K), jnp.float32)
        rr = r
        for j in range(_S):
            mj = jnp.max(rr, axis=1, keepdims=True)
            idxj = jnp.min(jnp.where(rr == mj, iota_k, _K), axis=1,
                           keepdims=True)
            oh = iota_k == idxj
            oh_sel = jnp.where(jrow == j, oh.astype(jnp.float32), oh_sel)
            rr = jnp.where(oh, -1.0, rr)
        mu_sel = jax.lax.dot_general(
            oh_sel, mu_ref[...], (((1,), (0,)), ((), ())),
            preferred_element_type=jnp.float32,
            precision=jax.lax.Precision.HIGHEST)
        ls_sel = jax.lax.dot_general(
            oh_sel, ls_ref[...], (((1,), (0,)), ((), ())),
            preferred_element_type=jnp.float32,
            precision=jax.lax.Precision.HIGHEST)
        out_ref[...] = mu_sel + jnp.exp(ls_sel) * eps_ref[...]


@jax.jit
def _run(x, mu, log_sigma, eps):
    out = pl.pallas_call(
        _router_body,
        grid=(_B,),
        in_specs=[
            pl.BlockSpec((1, _TH, _D), lambda b: (b, 0, 0)),
            pl.BlockSpec((1, _TH, _D), lambda b: (b, 1, 0)),
            pl.BlockSpec((1, _TH, _D), lambda b: (b, 2, 0)),
            pl.BlockSpec((1, _TH, _D), lambda b: (b, 3, 0)),
            pl.BlockSpec((_K, _D), lambda b: (0, 0)),
            pl.BlockSpec((_K, _D), lambda b: (0, 0)),
            pl.BlockSpec((_R, _D), lambda b: (0, 0)),
        ],
        out_specs=pl.BlockSpec((_R, _D), lambda b: (0, 0)),
        out_shape=jax.ShapeDtypeStruct((_R, _D), jnp.float32),
        scratch_shapes=[pltpu.VMEM((_R, _K), jnp.float32)],
        compiler_params=pltpu.CompilerParams(
            dimension_semantics=("arbitrary",)),
    )(x, x, x, x, mu, log_sigma, eps)
    return out.reshape(_B, _S, _D)


def kernel(x, mu, log_sigma, n_slots):
    eps = jax.random.normal(jax.random.key(42), (_R, _D), jnp.float32)
    return _run(x, mu, log_sigma, eps)
